# baseline (speedup 1.0000x reference)
import functools
import itertools

import numpy as np
import jax
import jax.numpy as jnp
from jax import lax
from jax.experimental import pallas as pl
from jax.experimental.pallas import tpu as pltpu
from jax.experimental.pallas import tpu_sc as plsc

HASH_SIZE = 2097152
NLEV = 8
GRIDS = np.round(np.geomspace(16, 2048, NLEV)).astype(np.int32)
N_PTS = 262144
NF = 2

NC = 2
NS = 16
L = 16
NW = NC * NS
PTS_PER_W = N_PTS // NW
C = 128
NCHUNK = PTS_PER_W // C
NG = C // L

_PI2 = jnp.int32(19349663)
_PI3 = jnp.int32(83492791)
_MASK = jnp.int32(HASH_SIZE - 1)


def _corner_data(xs, level):
    gf = float(GRIDS[level])
    ii, cw, fw = [], [], []
    for d in range(3):
        xn = (xs[d] + 2.0) * 0.25
        lo = xn * gf - 0.5
        iv = lo.astype(jnp.int32)
        fl = iv.astype(jnp.float32)
        ii.append(iv)
        cw.append(lo - fl)
        fw.append(1.0 - cw[-1])
    b2 = ii[1] * _PI2
    b2c = b2 + _PI2
    b3 = ii[2] * _PI3
    b3c = b3 + _PI3
    x0 = ii[0]
    x0c = x0 + 1
    base = jnp.int32(level * HASH_SIZE)
    wx = (fw[0], cw[0])
    wyz = {}
    for cy in range(2):
        for cz in range(2):
            wyz[(cy, cz)] = (fw[1] if cy == 0 else cw[1]) * (fw[2] if cz == 0 else cw[2])
    idx8, w8 = [], []
    for cx, cy, cz in itertools.product((0, 1), repeat=3):
        h = (x0c if cx else x0) ^ (b2c if cy else b2) ^ (b3c if cz else b3)
        idx8.append((h & _MASK) + base)
        w8.append(wx[cx] * wyz[(cy, cz)])
    return idx8, w8


def _body(xt_hbm, tab_hbm, out_hbm, xch, idxb, wbuf, gbuf, och, sem):
    wid = lax.axis_index("s") * NC + lax.axis_index("c")
    base0 = wid * PTS_PER_W

    def chunk_body(ci, carry):
        base = base0 + ci * C
        pltpu.sync_copy(xt_hbm.at[:, pl.ds(base, C)], xch)

        def group_a(g, c2):
            p = g * L
            xs = [xch[d, pl.ds(p, L)] for d in range(3)]
            for lev in range(NLEV):
                idx8, w8 = _corner_data(xs, lev)
                for cn in range(8):
                    lc = lev * 8 + cn
                    w0 = idx8[cn] * 2
                    idxb[lc, 0, pl.ds(p, L)] = w0
                    idxb[lc, 1, pl.ds(p, L)] = w0 + 1
                    wbuf[lc, pl.ds(p, L)] = w8[cn]
            return c2

        lax.fori_loop(0, NG, group_a, 0)

        copies = []
        for lc in range(NLEV * 8):
            for j in range(2):
                copies.append(
                    pltpu.async_copy(tab_hbm.at[idxb.at[lc, j]], gbuf.at[lc, j], sem))
        for cp in copies:
            cp.wait()

        def group_b(g, c2):
            p = g * L
            for lev in range(NLEV):
                acc0 = jnp.zeros((L,), jnp.float32)
                acc1 = jnp.zeros((L,), jnp.float32)
                for cn in range(8):
                    lc = lev * 8 + cn
                    w = wbuf[lc, pl.ds(p, L)]
                    acc0 = acc0 + w * gbuf[lc, 0, pl.ds(p, L)]
                    acc1 = acc1 + w * gbuf[lc, 1, pl.ds(p, L)]
                och[2 * lev, pl.ds(p, L)] = acc0 * 10.0
                och[2 * lev + 1, pl.ds(p, L)] = acc1 * 10.0
            return c2

        lax.fori_loop(0, NG, group_b, 0)
        pltpu.sync_copy(och, out_hbm.at[:, pl.ds(base, C)])
        return carry

    lax.fori_loop(0, NCHUNK, chunk_body, 0)


def _run(xt, tab):
    mesh = plsc.VectorSubcoreMesh(core_axis_name="c", subcore_axis_name="s")
    f = pl.kernel(
        _body,
        out_type=jax.ShapeDtypeStruct((2 * NLEV, N_PTS), jnp.float32),
        mesh=mesh,
        scratch_types=[
            pltpu.VMEM((3, C), jnp.float32),
            pltpu.VMEM((NLEV * 8, NF, C), jnp.int32),
            pltpu.VMEM((NLEV * 8, C), jnp.float32),
            pltpu.VMEM((NLEV * 8, NF, C), jnp.float32),
            pltpu.VMEM((2 * NLEV, C), jnp.float32),
            pltpu.SemaphoreType.DMA,
        ],
    )
    return f(xt, tab)


@jax.jit
def _encode(x, hash_tables):
    xt = x.T
    tab = hash_tables.reshape(NLEV * HASH_SIZE * NF)
    out_t = _run(xt, tab)
    return out_t.T


def kernel(x, hash_tables):
    return _encode(x, hash_tables)


# speedup vs baseline: 2.0326x; 2.0326x over previous
import functools
import itertools

import numpy as np
import jax
import jax.numpy as jnp
from jax import lax
from jax.experimental import pallas as pl
from jax.experimental.pallas import tpu as pltpu
from jax.experimental.pallas import tpu_sc as plsc

HASH_SIZE = 2097152
NLEV = 8
GRIDS = np.round(np.geomspace(16, 2048, NLEV)).astype(np.int32)
N_PTS = 262144
NF = 2

NC = 2
NS = 16
L = 16
NW = NC * NS
PTS_PER_W = N_PTS // NW
C = 128
NCHUNK = PTS_PER_W // C
NG = C // L

_PI2 = np.int32(19349663)
_PI3 = np.int32(83492791)
_MASK = np.int32(HASH_SIZE - 1)


def _dim_weights(xs, level):
    gf = float(GRIDS[level])
    cw, fw = [], []
    for d in range(3):
        xn = (xs[d] + 2.0) * 0.25
        lo = xn * gf - 0.5
        iv = lo.astype(jnp.int32)
        fl = iv.astype(jnp.float32)
        cw.append(lo - fl)
        fw.append(1.0 - cw[-1])
    return cw, fw


def _corner_rows(xs, level):
    gf = float(GRIDS[level])
    ii = []
    for d in range(3):
        xn = (xs[d] + 2.0) * 0.25
        lo = xn * gf - 0.5
        ii.append(lo.astype(jnp.int32))
    b2 = ii[1] * _PI2
    b2c = b2 + _PI2
    b3 = ii[2] * _PI3
    b3c = b3 + _PI3
    x0 = ii[0]
    x0c = x0 + 1
    rows = []
    for cx, cy, cz in itertools.product((0, 1), repeat=2 + 1):
        h = (x0c if cx else x0) ^ (b2c if cy else b2) ^ (b3c if cz else b3)
        rows.append(h & _MASK)
    return rows


def _body(xt_hbm, *refs):
    tabs = refs[:NLEV]
    out_hbm = refs[NLEV]
    xch, idxb, gbuf, och, sem = refs[NLEV + 1:]
    wid = lax.axis_index("s") * NC + lax.axis_index("c")
    base0 = wid * PTS_PER_W

    def chunk_body(ci, carry):
        base = base0 + ci * C
        pltpu.sync_copy(xt_hbm.at[:, pl.ds(base, C)], xch)

        def group_a(g, c2):
            p = g * L
            xs = [xch[d, pl.ds(p, L)] for d in range(3)]
            for lev in range(NLEV):
                rows = _corner_rows(xs, lev)
                for cn in range(8):
                    w0 = rows[cn] * 2
                    idxb[lev * 8 + cn, 0, pl.ds(p, L)] = w0
                    idxb[lev * 8 + cn, 1, pl.ds(p, L)] = w0 + 1
            return c2

        lax.fori_loop(0, NG, group_a, 0)

        copies = []
        for lev in range(NLEV):
            for cn in range(8):
                lc = lev * 8 + cn
                for f in range(NF):
                    copies.append(
                        pltpu.async_copy(tabs[lev].at[idxb.at[lc, f]],
                                         gbuf.at[lc, f], sem))
        for cp in copies:
            cp.wait()

        def group_b(g, c2):
            p = g * L
            xs = [xch[d, pl.ds(p, L)] for d in range(3)]
            for lev in range(NLEV):
                cw, fw = _dim_weights(xs, lev)
                wyz = {}
                for cy in range(2):
                    for cz in range(2):
                        wyz[(cy, cz)] = ((fw[1] if cy == 0 else cw[1])
                                         * (fw[2] if cz == 0 else cw[2]))
                acc0 = None
                acc1 = None
                for ci2, (cx, cy, cz) in enumerate(
                        itertools.product((0, 1), repeat=3)):
                    w = (fw[0] if cx == 0 else cw[0]) * wyz[(cy, cz)]
                    g0 = gbuf[lev * 8 + ci2, 0, pl.ds(p, L)]
                    g1 = gbuf[lev * 8 + ci2, 1, pl.ds(p, L)]
                    acc0 = w * g0 if acc0 is None else acc0 + w * g0
                    acc1 = w * g1 if acc1 is None else acc1 + w * g1
                och[2 * lev, pl.ds(p, L)] = acc0 * 10.0
                och[2 * lev + 1, pl.ds(p, L)] = acc1 * 10.0
            return c2

        lax.fori_loop(0, NG, group_b, 0)

        pltpu.sync_copy(och, out_hbm.at[:, pl.ds(base, C)])
        return carry

    lax.fori_loop(0, NCHUNK, chunk_body, 0)


def _run(xt, *tabs):
    mesh = plsc.VectorSubcoreMesh(core_axis_name="c", subcore_axis_name="s")
    f = pl.kernel(
        _body,
        out_type=jax.ShapeDtypeStruct((2 * NLEV, N_PTS), jnp.float32),
        mesh=mesh,
        scratch_types=[
            pltpu.VMEM((3, C), jnp.float32),
            pltpu.VMEM((NLEV * 8, NF, C), jnp.int32),
            pltpu.VMEM((NLEV * 8, NF, C), jnp.float32),
            pltpu.VMEM((2 * NLEV, C), jnp.float32),
            pltpu.SemaphoreType.DMA,
        ],
    )
    return f(xt, *tabs)


@jax.jit
def _encode(x, hash_tables):
    xt = x.T
    tabs = [hash_tables[i].reshape(HASH_SIZE * NF) for i in range(NLEV)]
    out = _run(xt, *tabs)
    return out.T


def kernel(x, hash_tables):
    return _encode(x, hash_tables)


# speedup vs baseline: 12.2469x; 6.0252x over previous
import functools
import itertools

import numpy as np
import jax
import jax.numpy as jnp
from jax import lax
from jax.experimental import pallas as pl
from jax.experimental.pallas import tpu as pltpu
from jax.experimental.pallas import tpu_sc as plsc

HASH_SIZE = 2097152
NLEV = 8
GRIDS = np.round(np.geomspace(16, 2048, NLEV)).astype(np.int32)
N_PTS = 262144
NF = 2

NC = 2
NS = 16
L = 16
NW = NC * NS
PTS_PER_W = N_PTS // NW
C = 128
NCHUNK = PTS_PER_W // C
NG = C // L

_PI2 = np.int32(19349663)
_PI3 = np.int32(83492791)
_MASK = np.int32(HASH_SIZE - 1)


def _dim_weights(xs, level):
    gf = float(GRIDS[level])
    cw, fw = [], []
    for d in range(3):
        xn = (xs[d] + 2.0) * 0.25
        lo = xn * gf - 0.5
        iv = lo.astype(jnp.int32)
        fl = iv.astype(jnp.float32)
        cw.append(lo - fl)
        fw.append(1.0 - cw[-1])
    return cw, fw


def _corner_rows(xs, level):
    gf = float(GRIDS[level])
    ii = []
    for d in range(3):
        xn = (xs[d] + 2.0) * 0.25
        lo = xn * gf - 0.5
        ii.append(lo.astype(jnp.int32))
    b2 = ii[1] * _PI2
    b2c = b2 + _PI2
    b3 = ii[2] * _PI3
    b3c = b3 + _PI3
    x0 = ii[0]
    x0c = x0 + 1
    rows = []
    for cx, cy, cz in itertools.product((0, 1), repeat=2 + 1):
        h = (x0c if cx else x0) ^ (b2c if cy else b2) ^ (b3c if cz else b3)
        rows.append(h & _MASK)
    return rows


def _body(xt_hbm, *refs):
    tabs = refs[:NLEV * NF]
    out_hbm = refs[NLEV * NF]
    xch, idxb, gbuf, och, sem = refs[NLEV * NF + 1:]
    wid = lax.axis_index("s") * NC + lax.axis_index("c")
    base0 = wid * PTS_PER_W

    def chunk_body(ci, carry):
        base = base0 + ci * C
        pltpu.sync_copy(xt_hbm.at[:, pl.ds(base, C)], xch)

        def group_a(g, c2):
            p = g * L
            xs = [xch[d, pl.ds(p, L)] for d in range(3)]
            for lev in range(NLEV):
                rows = _corner_rows(xs, lev)
                for cn in range(8):
                    idxb[lev * 8 + cn, pl.ds(p, L)] = rows[cn]
            return c2

        lax.fori_loop(0, NG, group_a, 0)

        copies = []
        for lev in range(NLEV):
            for cn in range(8):
                lc = lev * 8 + cn
                for f in range(NF):
                    copies.append(
                        pltpu.async_copy(tabs[lev * NF + f].at[idxb.at[lc]],
                                         gbuf.at[lc, f], sem))
        for cp in copies:
            cp.wait()

        def group_b(g, c2):
            p = g * L
            xs = [xch[d, pl.ds(p, L)] for d in range(3)]
            for lev in range(NLEV):
                cw, fw = _dim_weights(xs, lev)
                wyz = {}
                for cy in range(2):
                    for cz in range(2):
                        wyz[(cy, cz)] = ((fw[1] if cy == 0 else cw[1])
                                         * (fw[2] if cz == 0 else cw[2]))
                acc0 = None
                acc1 = None
                for ci2, (cx, cy, cz) in enumerate(
                        itertools.product((0, 1), repeat=3)):
                    w = (fw[0] if cx == 0 else cw[0]) * wyz[(cy, cz)]
                    g0 = gbuf[lev * 8 + ci2, 0, pl.ds(p, L)]
                    g1 = gbuf[lev * 8 + ci2, 1, pl.ds(p, L)]
                    acc0 = w * g0 if acc0 is None else acc0 + w * g0
                    acc1 = w * g1 if acc1 is None else acc1 + w * g1
                och[2 * lev, pl.ds(p, L)] = acc0 * 10.0
                och[2 * lev + 1, pl.ds(p, L)] = acc1 * 10.0
            return c2

        lax.fori_loop(0, NG, group_b, 0)

        pltpu.sync_copy(och, out_hbm.at[:, pl.ds(base, C)])
        return carry

    lax.fori_loop(0, NCHUNK, chunk_body, 0)


def _run(xt, *tabs):
    mesh = plsc.VectorSubcoreMesh(core_axis_name="c", subcore_axis_name="s")
    f = pl.kernel(
        _body,
        out_type=jax.ShapeDtypeStruct((2 * NLEV, N_PTS), jnp.float32),
        mesh=mesh,
        scratch_types=[
            pltpu.VMEM((3, C), jnp.float32),
            pltpu.VMEM((NLEV * 8, C), jnp.int32),
            pltpu.VMEM((NLEV * 8, NF, C), jnp.float32),
            pltpu.VMEM((2 * NLEV, C), jnp.float32),
            pltpu.SemaphoreType.DMA,
        ],
    )
    return f(xt, *tabs)


@jax.jit
def _encode(x, hash_tables):
    xt = x.T
    tabs = [hash_tables[i, :, f] for i in range(NLEV) for f in range(NF)]
    out = _run(xt, *tabs)
    return out.T


def kernel(x, hash_tables):
    return _encode(x, hash_tables)
